# baseline (speedup 1.0000x reference)
import functools
import math

import jax
import jax.numpy as jnp
from jax import lax
from jax.experimental import pallas as pl
from jax.experimental.pallas import tpu as pltpu
from jax.experimental.pallas import tpu_sc as plsc

D = 64
NW = 32
CHUNK = 128
NBUF = 2
SCALE = 1.0 / math.sqrt(D)


def _make_emb_kernel(b_total: int):
    b_per_w = b_total // NW
    n_chunks = b_per_w // CHUNK
    mesh = plsc.VectorSubcoreMesh(core_axis_name="c", subcore_axis_name="s")

    @functools.partial(
        pl.kernel,
        out_type=jax.ShapeDtypeStruct((b_total // 2, 2 * D), jnp.float32),
        mesh=mesh,
        scratch_types=[
            pltpu.VMEM((n_chunks, CHUNK), jnp.int32),
            pltpu.VMEM((n_chunks, CHUNK), jnp.int32),
            [pltpu.VMEM((CHUNK, 2 * D), jnp.float32) for _ in range(NBUF)],
            [pltpu.VMEM((CHUNK // 2, 2 * D), jnp.float32) for _ in range(NBUF)],
            [pltpu.SemaphoreType.DMA for _ in range(NBUF)],
            [pltpu.SemaphoreType.DMA for _ in range(NBUF)],
        ],
    )
    def emb(x_hbm, table_hbm, out_hbm, idx_all, pidx_all, rows, srows,
            gsems, osems):
        wid = lax.axis_index("s") * 2 + lax.axis_index("c")
        base2 = wid * (b_per_w // 2)

        pltpu.sync_copy(x_hbm.at[wid], idx_all)

        def shift_row(j):
            for c in range(0, CHUNK, 16):
                pidx_all[j, pl.ds(c, 16)] = (
                    idx_all[j, pl.ds(c, 16)] >> 1
                )

        pl.loop(0, n_chunks)(shift_row)

        for b in range(NBUF):
            pltpu.async_copy(table_hbm.at[pidx_all.at[b]], rows[b], gsems[b])

        def chunk_pair(i0):
            for b in range(NBUF):
                i = i0 + b
                pltpu.make_async_copy(
                    table_hbm.at[pidx_all.at[i]], rows[b], gsems[b]
                ).wait()

                @pl.when(i >= NBUF)
                def _():
                    pltpu.make_async_copy(
                        srows[b], out_hbm.at[pl.ds(0, CHUNK // 2)], osems[b]
                    ).wait()

                def scale_group(g):
                    iv = idx_all[i, pl.ds(g * 16, 16)]
                    pv = (iv & 1) * D
                    for j in range(16):
                        r = g * 16 + j
                        p = pv[j]
                        r2 = g * 8 + j // 2
                        half = (j % 2) * D
                        for c in range(0, D, 16):
                            srows[b][r2, pl.ds(half + c, 16)] = (
                                rows[b][r, pl.ds(p + c, 16)] * SCALE
                            )

                pl.loop(0, CHUNK // 16)(scale_group)

                @pl.when(i + NBUF < n_chunks)
                def _():
                    pltpu.async_copy(
                        table_hbm.at[pidx_all.at[i + NBUF]], rows[b], gsems[b]
                    )

                pltpu.async_copy(
                    srows[b],
                    out_hbm.at[pl.ds(base2 + i * (CHUNK // 2), CHUNK // 2)],
                    osems[b],
                )

        pl.loop(0, n_chunks, step=NBUF)(chunk_pair)

        for b in range(NBUF):
            pltpu.make_async_copy(
                srows[b], out_hbm.at[pl.ds(0, CHUNK // 2)], osems[b]
            ).wait()

    return emb


def kernel(x, table):
    b, s = x.shape
    b_total = b * s
    x_grouped = x.reshape(NW, (b_total // NW) // CHUNK, CHUNK).astype(jnp.int32)
    table2 = table.reshape(table.shape[0] // 2, 2 * D)
    out2 = _make_emb_kernel(b_total)(x_grouped, table2)
    return out2.reshape(b, s, D)


# speedup vs baseline: 1.5698x; 1.5698x over previous
import functools
import math

import jax
import jax.numpy as jnp
from jax import lax
from jax.experimental import pallas as pl
from jax.experimental.pallas import tpu as pltpu
from jax.experimental.pallas import tpu_sc as plsc

D = 64
NW = 32
CHUNK = 128
NG = 4
NS = 2
SCALE = 1.0 / math.sqrt(D)


def _make_emb_kernel(b_total: int):
    b_per_w = b_total // NW
    n_chunks = b_per_w // CHUNK
    mesh = plsc.VectorSubcoreMesh(core_axis_name="c", subcore_axis_name="s")

    @functools.partial(
        pl.kernel,
        out_type=jax.ShapeDtypeStruct((b_total, D), jnp.float32),
        mesh=mesh,
        scratch_types=[
            pltpu.VMEM((n_chunks, CHUNK), jnp.int32),
            [pltpu.VMEM((CHUNK, 2 * D), jnp.float32) for _ in range(NG)],
            [pltpu.VMEM((CHUNK, D), jnp.float32) for _ in range(NS)],
            [pltpu.SemaphoreType.DMA for _ in range(NG)],
            [pltpu.SemaphoreType.DMA for _ in range(NS)],
        ],
    )
    def emb(x_hbm, table_hbm, out_hbm, idx_all, rows, srows, gsems, osems):
        wid = lax.axis_index("s") * 2 + lax.axis_index("c")
        base = wid * b_per_w

        pltpu.sync_copy(x_hbm.at[wid], idx_all)

        for g in range(NG):
            pltpu.async_copy(table_hbm.at[idx_all.at[g]], rows[g], gsems[g])

        def chunk_group(i0):
            for k in range(NG):
                i = i0 + k
                g = k % NG
                s = k % NS
                pltpu.make_async_copy(
                    table_hbm.at[idx_all.at[i]], rows[g], gsems[g]
                ).wait()

                @pl.when(i >= NS)
                def _():
                    pltpu.make_async_copy(
                        srows[s], out_hbm.at[pl.ds(0, CHUNK)], osems[s]
                    ).wait()

                def scale_row(r):
                    for c in range(0, D, 16):
                        srows[s][r, pl.ds(c, 16)] = (
                            rows[g][r, pl.ds(c, 16)] * SCALE
                        )

                plsc.parallel_loop(0, CHUNK, unroll=2)(scale_row)

                @pl.when(i + NG < n_chunks)
                def _():
                    pltpu.async_copy(
                        table_hbm.at[idx_all.at[i + NG]], rows[g], gsems[g]
                    )

                pltpu.async_copy(
                    srows[s],
                    out_hbm.at[pl.ds(base + i * CHUNK, CHUNK)],
                    osems[s],
                )

        pl.loop(0, n_chunks, step=NG)(chunk_group)

        for s in range(NS):
            pltpu.make_async_copy(
                srows[s], out_hbm.at[pl.ds(0, CHUNK)], osems[s]
            ).wait()

    return emb


def kernel(x, table):
    nb, s = x.shape
    b_total = nb * s
    x_grouped = x.reshape(NW, (b_total // NW) // CHUNK, CHUNK).astype(jnp.int32)
    table_p = jnp.pad(table, ((0, 0), (0, D)))
    out = _make_emb_kernel(b_total)(x_grouped, table_p)
    return out.reshape(nb, s, D)
